# baseline (speedup 1.0000x reference)
import jax
import jax.numpy as jnp
from jax import lax
from jax.experimental import pallas as pl
from jax.experimental.pallas import tpu as pltpu
from jax.experimental.pallas import tpu_sc as plsc

N_USERS = 10000
N_ITEMS = 10000
N_EDGES = 320000
D = 128

NC = 2
NS = 16
L = 16

EPS = N_EDGES // NS
BLK = 80
NB = 256
EPSP = NB * BLK
PAD = EPSP - EPS
ROWS_PER_SUB = 624
ROWS_TAIL = N_USERS - NS * ROWS_PER_SUB


def _body(tab, gidx, sidx, norm, out_u, out_i, acc,
          gi0, gi1, gi2, gi3, si0, si1, si2, si3, nm0, nm1, nm2, nm3,
          rows0, rows1, rows2, rows3, sg0, sg1, sg2, sg3,
          ss0, ss1, ss2, ss3, sv0, sv1, sv2, sv3):
    c = lax.axis_index("c")
    s = lax.axis_index("s")
    rows = (rows0, rows1, rows2, rows3)
    gi = (gi0, gi1, gi2, gi3)
    si = (si0, si1, si2, si3)
    nm = (nm0, nm1, nm2, nm3)
    sem_g = (sg0, sg1, sg2, sg3)
    sem_s = (ss0, ss1, ss2, ss3)
    sem_i = (sv0, sv1, sv2, sv3)
    sbase = s * EPSP
    cbase = c * NS * EPSP + sbase

    def idx_copies(bq, q):
        return (
            pltpu.make_async_copy(gidx.at[pl.ds(cbase + bq * BLK, BLK)],
                                  gi[q], sem_i[q]),
            pltpu.make_async_copy(sidx.at[pl.ds(cbase + bq * BLK, BLK)],
                                  si[q], sem_i[q]),
            pltpu.make_async_copy(norm.at[pl.ds(sbase + bq * BLK, BLK)],
                                  nm[q], sem_i[q]),
        )

    def zero_row(r, _):
        for k in range(D // L):
            rows3[r, pl.ds(k * L, L)] = jnp.zeros((L,), jnp.float32)
        return 0
    lax.fori_loop(0, BLK, zero_row, 0)

    base_row = s * ROWS_PER_SUB
    for j in range(ROWS_PER_SUB // BLK):
        pltpu.sync_copy(rows3, acc.at[pl.ds(base_row + j * BLK, BLK)])
    rem = ROWS_PER_SUB % BLK
    pltpu.sync_copy(rows3.at[pl.ds(0, rem)],
                    acc.at[pl.ds(base_row + ROWS_PER_SUB - rem, rem)])

    @pl.when(s == 0)
    def _():
        pltpu.sync_copy(rows3.at[pl.ds(0, ROWS_TAIL)],
                        acc.at[pl.ds(NS * ROWS_PER_SUB, ROWS_TAIL)])

    plsc.subcore_barrier()

    for q in range(3):
        for cp in idx_copies(q, q):
            cp.start()
    for q in range(3):
        for cp in idx_copies(q, q):
            cp.wait()
        pltpu.async_copy(tab.at[gi[q]], rows[q], sem_g[q])
    pltpu.async_copy(rows3, acc.at[si[0]], sem_s[3], add=True)

    def iter4(t, _):
        for k in range(4):
            b = 4 * t + k
            kn = (k + 3) % 4
            pltpu.make_async_copy(rows[kn], acc.at[si[kn]],
                                  sem_s[kn]).wait()
            bw3 = jnp.where(b + 3 < NB, b + 3, b + 3 - NB)
            for cp in idx_copies(bw3, kn):
                cp.start()
            pltpu.make_async_copy(tab.at[gi[k]], rows[k], sem_g[k]).wait()
            for cp in idx_copies(bw3, kn):
                cp.wait()
            pltpu.async_copy(tab.at[gi[kn]], rows[kn], sem_g[kn])
            def scale_grp(g, _, k=k):
                for j in range(L):
                    r = g * L + j
                    sc = plsc.load_gather(nm[k],
                                          [jnp.full((L,), r, jnp.int32)])
                    for kk in range(D // L):
                        rows[k][r, pl.ds(kk * L, L)] = (
                            rows[k][r, pl.ds(kk * L, L)] * sc)
                return 0
            lax.fori_loop(0, BLK // L, scale_grp, 0)
            pltpu.async_copy(rows[k], acc.at[si[k]], sem_s[k], add=True)
        return 0
    lax.fori_loop(0, NB // 4, iter4, 0)

    for q in range(3):
        pltpu.make_async_copy(tab.at[gi[q]], rows[q], sem_g[q]).wait()
    pltpu.make_async_copy(rows3, acc.at[si[3]], sem_s[3]).wait()

    plsc.subcore_barrier()

    @pl.when(c == 0)
    def _():
        pltpu.sync_copy(acc.at[pl.ds(base_row, ROWS_PER_SUB)],
                        out_i.at[pl.ds(base_row, ROWS_PER_SUB)])

        @pl.when(s == 0)
        def _():
            pltpu.sync_copy(acc.at[pl.ds(NS * ROWS_PER_SUB, ROWS_TAIL)],
                            out_i.at[pl.ds(NS * ROWS_PER_SUB, ROWS_TAIL)])

    @pl.when(c == 1)
    def _():
        pltpu.sync_copy(acc.at[pl.ds(base_row, ROWS_PER_SUB)],
                        out_u.at[pl.ds(base_row, ROWS_PER_SUB)])

        @pl.when(s == 0)
        def _():
            pltpu.sync_copy(acc.at[pl.ds(NS * ROWS_PER_SUB, ROWS_TAIL)],
                            out_u.at[pl.ds(NS * ROWS_PER_SUB, ROWS_TAIL)])


@jax.jit
def kernel(user_emb, item_emb, edge_index, edge_norm):
    u = edge_index[0].astype(jnp.int32)
    i = edge_index[1].astype(jnp.int32)
    tab = jnp.concatenate([user_emb, item_emb], axis=0)

    def pad16(x):
        return jnp.pad(x.reshape(NS, EPS), ((0, 0), (0, PAD))).reshape(-1)

    gidx = jnp.concatenate([pad16(u), pad16(i + N_USERS)])
    sidx = jnp.concatenate([pad16(i), pad16(u)])
    norm = pad16(edge_norm)

    mesh = plsc.VectorSubcoreMesh(core_axis_name="c", subcore_axis_name="s",
                                  num_cores=NC, num_subcores=NS)
    run = pl.kernel(
        _body,
        out_type=(jax.ShapeDtypeStruct((N_USERS, D), jnp.float32),
                  jax.ShapeDtypeStruct((N_ITEMS, D), jnp.float32)),
        mesh=mesh,
        compiler_params=pltpu.CompilerParams(needs_layout_passes=False),
        scratch_types=(
            [pltpu.VMEM_SHARED((N_USERS, D), jnp.float32)]
            + [pltpu.VMEM((BLK,), jnp.int32) for _ in range(4)]
            + [pltpu.VMEM((BLK,), jnp.int32) for _ in range(4)]
            + [pltpu.VMEM((BLK,), jnp.float32) for _ in range(4)]
            + [pltpu.VMEM((BLK, D), jnp.float32) for _ in range(4)]
            + [pltpu.SemaphoreType.DMA for _ in range(12)]
        ),
    )
    agg_users, agg_items = run(tab, gidx, sidx, norm)
    return (agg_users, agg_items)


# speedup vs baseline: 1.1236x; 1.1236x over previous
import jax
import jax.numpy as jnp
from jax import lax
from jax.experimental import pallas as pl
from jax.experimental.pallas import tpu as pltpu
from jax.experimental.pallas import tpu_sc as plsc

N_USERS = 10000
N_ITEMS = 10000
N_EDGES = 320000
D = 128

NC = 2
NS = 16
L = 16

EPS = N_EDGES // NS
BLK = 128
NB = 160
EPSP = NB * BLK
PAD = EPSP - EPS
ROWS_PER_SUB = 624
ROWS_TAIL = N_USERS - NS * ROWS_PER_SUB


def _body(tab, gidx, sidx, norm, out_u, out_i, acc,
          gi0, gi1, gi2, gi3, si0, si1, si2, si3, nm0, nm1, nm2, nm3,
          rows0, rows1, sg0, sg1, ss0, ss1, si_sem0, si_sem1, si_sem2,
          si_sem3):
    c = lax.axis_index("c")
    s = lax.axis_index("s")
    rows = (rows0, rows1)
    gi = (gi0, gi1, gi2, gi3)
    si = (si0, si1, si2, si3)
    nm = (nm0, nm1, nm2, nm3)
    sem_g = (sg0, sg1)
    sem_s = (ss0, ss1)
    sem_i = (si_sem0, si_sem1, si_sem2, si_sem3)
    sbase = s * EPSP
    cbase = c * NS * EPSP + sbase

    def idx_copies(bq, q):
        return (
            pltpu.make_async_copy(gidx.at[pl.ds(cbase + bq * BLK, BLK)],
                                  gi[q], sem_i[q]),
            pltpu.make_async_copy(sidx.at[pl.ds(cbase + bq * BLK, BLK)],
                                  si[q], sem_i[q]),
            pltpu.make_async_copy(norm.at[pl.ds(sbase + bq * BLK, BLK)],
                                  nm[q], sem_i[q]),
        )

    def zero_row(r, _):
        for k in range(D // L):
            z = jnp.zeros((L,), jnp.float32)
            rows0[r, pl.ds(k * L, L)] = z
            rows1[r, pl.ds(k * L, L)] = z
        return 0
    lax.fori_loop(0, BLK, zero_row, 0)

    base_row = s * ROWS_PER_SUB
    for j in range(ROWS_PER_SUB // BLK):
        pltpu.sync_copy(rows0, acc.at[pl.ds(base_row + j * BLK, BLK)])
    rem = ROWS_PER_SUB % BLK
    pltpu.sync_copy(rows0.at[pl.ds(0, rem)],
                    acc.at[pl.ds(base_row + ROWS_PER_SUB - rem, rem)])

    @pl.when(s == 0)
    def _():
        pltpu.sync_copy(rows0.at[pl.ds(0, ROWS_TAIL)],
                        acc.at[pl.ds(NS * ROWS_PER_SUB, ROWS_TAIL)])

    plsc.subcore_barrier()

    for cp in idx_copies(0, 0):
        cp.start()
    for cp in idx_copies(1, 1):
        cp.start()
    for cp in idx_copies(0, 0):
        cp.wait()
    pltpu.async_copy(rows1, acc.at[si[0]], sem_s[1], add=True)
    pltpu.async_copy(tab.at[gi[0]], rows0, sem_g[0])

    def iter4(t, _):
        for k in range(4):
            b = 4 * t + k
            p = k % 2
            bw2 = jnp.where(b + 2 < NB, b + 2, b + 2 - NB)
            for cp in idx_copies(bw2, (k + 2) % 4):
                cp.start()
            pltpu.make_async_copy(tab.at[gi[k]], rows[p], sem_g[p]).wait()
            pltpu.make_async_copy(rows[1 - p], acc.at[si[(k + 3) % 4]],
                                  sem_s[1 - p]).wait()
            bw1 = jnp.where(b + 1 < NB, b + 1, 0)
            for cp in idx_copies(bw1, (k + 1) % 4):
                cp.wait()
            pltpu.async_copy(tab.at[gi[(k + 1) % 4]], rows[1 - p],
                             sem_g[1 - p])
            def scale_grp(g, _, p=p, k=k):
                for j in range(L):
                    r = g * L + j
                    sc = plsc.load_gather(nm[k],
                                          [jnp.full((L,), r, jnp.int32)])
                    for kk in range(D // L):
                        rows[p][r, pl.ds(kk * L, L)] = (
                            rows[p][r, pl.ds(kk * L, L)] * sc)
                return 0
            lax.fori_loop(0, BLK // L, scale_grp, 0)
            pltpu.async_copy(rows[p], acc.at[si[k]], sem_s[p], add=True)
        return 0
    lax.fori_loop(0, NB // 4, iter4, 0)

    pltpu.make_async_copy(tab.at[gi[0]], rows0, sem_g[0]).wait()
    pltpu.make_async_copy(rows1, acc.at[si[3]], sem_s[1]).wait()
    for cp in idx_copies(1, 1):
        cp.wait()

    plsc.subcore_barrier()

    @pl.when(c == 0)
    def _():
        pltpu.sync_copy(acc.at[pl.ds(base_row, ROWS_PER_SUB)],
                        out_i.at[pl.ds(base_row, ROWS_PER_SUB)])

        @pl.when(s == 0)
        def _():
            pltpu.sync_copy(acc.at[pl.ds(NS * ROWS_PER_SUB, ROWS_TAIL)],
                            out_i.at[pl.ds(NS * ROWS_PER_SUB, ROWS_TAIL)])

    @pl.when(c == 1)
    def _():
        pltpu.sync_copy(acc.at[pl.ds(base_row, ROWS_PER_SUB)],
                        out_u.at[pl.ds(base_row, ROWS_PER_SUB)])

        @pl.when(s == 0)
        def _():
            pltpu.sync_copy(acc.at[pl.ds(NS * ROWS_PER_SUB, ROWS_TAIL)],
                            out_u.at[pl.ds(NS * ROWS_PER_SUB, ROWS_TAIL)])


@jax.jit
def kernel(user_emb, item_emb, edge_index, edge_norm):
    u = edge_index[0].astype(jnp.int32)
    i = edge_index[1].astype(jnp.int32)
    tab = jnp.concatenate([user_emb, item_emb], axis=0)

    def pad16(x):
        return jnp.pad(x.reshape(NS, EPS), ((0, 0), (0, PAD))).reshape(-1)

    gidx = jnp.concatenate([pad16(u), pad16(i + N_USERS)])
    sidx = jnp.concatenate([pad16(i), pad16(u)])
    norm = pad16(edge_norm)

    mesh = plsc.VectorSubcoreMesh(core_axis_name="c", subcore_axis_name="s",
                                  num_cores=NC, num_subcores=NS)
    run = pl.kernel(
        _body,
        out_type=(jax.ShapeDtypeStruct((N_USERS, D), jnp.float32),
                  jax.ShapeDtypeStruct((N_ITEMS, D), jnp.float32)),
        mesh=mesh,
        compiler_params=pltpu.CompilerParams(needs_layout_passes=False),
        scratch_types=(
            [pltpu.VMEM_SHARED((N_USERS, D), jnp.float32)]
            + [pltpu.VMEM((BLK,), jnp.int32) for _ in range(4)]
            + [pltpu.VMEM((BLK,), jnp.int32) for _ in range(4)]
            + [pltpu.VMEM((BLK,), jnp.float32) for _ in range(4)]
            + [pltpu.VMEM((BLK, D), jnp.float32) for _ in range(2)]
            + [pltpu.SemaphoreType.DMA for _ in range(8)]
        ),
    )
    agg_users, agg_items = run(tab, gidx, sidx, norm)
    return (agg_users, agg_items)


# speedup vs baseline: 1.3527x; 1.2039x over previous
import functools

import jax
import jax.numpy as jnp
from jax import lax
from jax.experimental import pallas as pl
from jax.experimental.pallas import tpu as pltpu
from jax.experimental.pallas import tpu_sc as plsc

N_USERS = 10000
N_ITEMS = 10000
N_EDGES = 320000
D = 128

NC = 2
NS = 16
L = 16

BLK = 128
NBLK = N_EDGES // BLK
ROWS_PER_SUB = 624
ROWS_TAIL = N_USERS - NS * ROWS_PER_SUB


def _body(tab, gidx, sidx, norm, out_u, out_i, acc, gi_v, si_v, nrm_v,
          rows_v, sem):
    c = lax.axis_index("c")
    s = lax.axis_index("s")

    def zero_row(r, _):
        for k in range(D // L):
            rows_v[r, pl.ds(k * L, L)] = jnp.zeros((L,), jnp.float32)
        return 0
    lax.fori_loop(0, BLK, zero_row, 0)
    base_row = s * ROWS_PER_SUB
    for j in range(4):
        pltpu.sync_copy(rows_v, acc.at[pl.ds(base_row + j * BLK, BLK)])
    pltpu.sync_copy(rows_v.at[pl.ds(0, ROWS_PER_SUB - 4 * BLK)],
                    acc.at[pl.ds(base_row + 4 * BLK, ROWS_PER_SUB - 4 * BLK)])

    @pl.when(s == 0)
    def _():
        pltpu.sync_copy(rows_v.at[pl.ds(0, ROWS_TAIL)],
                        acc.at[pl.ds(NS * ROWS_PER_SUB, ROWS_TAIL)])
    plsc.subcore_barrier()

    nblk_mine = jnp.where(s < (NBLK % NS), NBLK // NS + 1, NBLK // NS)

    def block(b, _):
        base = (s + b * NS) * BLK
        pltpu.sync_copy(gidx.at[c, pl.ds(base, BLK)], gi_v)
        pltpu.sync_copy(sidx.at[c, pl.ds(base, BLK)], si_v)
        pltpu.sync_copy(norm.at[pl.ds(base, BLK)], nrm_v)
        pltpu.async_copy(tab.at[gi_v], rows_v, sem).wait()

        def scale_row(r, _):
            sc = plsc.load_gather(nrm_v, [jnp.full((L,), r, jnp.int32)])
            for k in range(D // L):
                rows_v[r, pl.ds(k * L, L)] = rows_v[r, pl.ds(k * L, L)] * sc
            return 0
        lax.fori_loop(0, BLK, scale_row, 0)

        pltpu.sync_copy(rows_v, acc.at[si_v], add=True)
        return 0
    lax.fori_loop(0, nblk_mine, block, 0)

    plsc.subcore_barrier()

    @pl.when(c == 0)
    def _():
        pltpu.sync_copy(acc.at[pl.ds(base_row, ROWS_PER_SUB)],
                        out_i.at[pl.ds(base_row, ROWS_PER_SUB)])

        @pl.when(s == 0)
        def _():
            pltpu.sync_copy(acc.at[pl.ds(NS * ROWS_PER_SUB, ROWS_TAIL)],
                            out_i.at[pl.ds(NS * ROWS_PER_SUB, ROWS_TAIL)])

    @pl.when(c == 1)
    def _():
        pltpu.sync_copy(acc.at[pl.ds(base_row, ROWS_PER_SUB)],
                        out_u.at[pl.ds(base_row, ROWS_PER_SUB)])

        @pl.when(s == 0)
        def _():
            pltpu.sync_copy(acc.at[pl.ds(NS * ROWS_PER_SUB, ROWS_TAIL)],
                            out_u.at[pl.ds(NS * ROWS_PER_SUB, ROWS_TAIL)])


@jax.jit
def kernel(user_emb, item_emb, edge_index, edge_norm):
    u = edge_index[0].astype(jnp.int32)
    i = edge_index[1].astype(jnp.int32)
    tab = jnp.concatenate([user_emb, item_emb], axis=0)
    gidx = jnp.stack([u, i + N_USERS], axis=0)
    sidx = jnp.stack([i, u], axis=0)

    mesh = plsc.VectorSubcoreMesh(core_axis_name="c", subcore_axis_name="s",
                                  num_cores=NC, num_subcores=NS)
    run = pl.kernel(
        _body,
        out_type=(jax.ShapeDtypeStruct((N_USERS, D), jnp.float32),
                  jax.ShapeDtypeStruct((N_ITEMS, D), jnp.float32)),
        mesh=mesh,
        compiler_params=pltpu.CompilerParams(needs_layout_passes=False),
        scratch_types=[
            pltpu.VMEM_SHARED((N_USERS, D), jnp.float32),
            pltpu.VMEM((BLK,), jnp.int32),
            pltpu.VMEM((BLK,), jnp.int32),
            pltpu.VMEM((BLK,), jnp.float32),
            pltpu.VMEM((BLK, D), jnp.float32),
            pltpu.SemaphoreType.DMA,
        ],
    )
    agg_users, agg_items = run(tab, gidx, sidx, edge_norm)
    return (agg_users, agg_items)


# speedup vs baseline: 1.4467x; 1.0695x over previous
import jax
import jax.numpy as jnp
from jax import lax
from jax.experimental import pallas as pl
from jax.experimental.pallas import tpu as pltpu
from jax.experimental.pallas import tpu_sc as plsc

N_USERS = 10000
N_ITEMS = 10000
N_EDGES = 320000
D = 128

NC = 2
NS = 16
L = 16

BLK = 128
SB = 2 * BLK
NSB = N_EDGES // SB
ROWS_PER_SUB = 624
ROWS_TAIL = N_USERS - NS * ROWS_PER_SUB


def _body(tab, gidx, sidx, norm, out_u, out_i, acc,
          gi0, gi1, si0, si1, nm0, nm1, rows0, rows1, sem0, sem1):
    c = lax.axis_index("c")
    s = lax.axis_index("s")
    gi = (gi0, gi1)
    si = (si0, si1)
    nm = (nm0, nm1)
    rows = (rows0, rows1)

    def zero_row(r, _):
        for k in range(D // L):
            rows0[r, pl.ds(k * L, L)] = jnp.zeros((L,), jnp.float32)
        return 0
    lax.fori_loop(0, BLK, zero_row, 0)
    base_row = s * ROWS_PER_SUB
    for j in range(4):
        pltpu.sync_copy(rows0, acc.at[pl.ds(base_row + j * BLK, BLK)])
    pltpu.sync_copy(rows0.at[pl.ds(0, ROWS_PER_SUB - 4 * BLK)],
                    acc.at[pl.ds(base_row + 4 * BLK, ROWS_PER_SUB - 4 * BLK)])

    @pl.when(s == 0)
    def _():
        pltpu.sync_copy(rows0.at[pl.ds(0, ROWS_TAIL)],
                        acc.at[pl.ds(NS * ROWS_PER_SUB, ROWS_TAIL)])
    plsc.subcore_barrier()

    nsb_mine = jnp.where(s < (NSB % NS), NSB // NS + 1, NSB // NS)

    def scale_half(h):
        def scale_row(r, _):
            sc = plsc.load_gather(nm[h], [jnp.full((L,), r, jnp.int32)])
            for k in range(D // L):
                rows[h][r, pl.ds(k * L, L)] = rows[h][r, pl.ds(k * L, L)] * sc
            return 0
        lax.fori_loop(0, BLK, scale_row, 0)

    def block(b, _):
        base = (s + b * NS) * SB
        for h in range(2):
            pltpu.sync_copy(gidx.at[c, pl.ds(base + h * BLK, BLK)], gi[h])
            pltpu.sync_copy(sidx.at[c, pl.ds(base + h * BLK, BLK)], si[h])
            pltpu.sync_copy(norm.at[pl.ds(base + h * BLK, BLK)], nm[h])
        g0 = pltpu.make_async_copy(tab.at[gi0], rows0, sem0)
        g1 = pltpu.make_async_copy(tab.at[gi1], rows1, sem1)
        g0.start()
        g1.start()
        g0.wait()
        scale_half(0)
        g1.wait()
        scale_half(1)
        pltpu.sync_copy(rows0, acc.at[si0], add=True)
        pltpu.sync_copy(rows1, acc.at[si1], add=True)
        return 0
    lax.fori_loop(0, nsb_mine, block, 0)

    plsc.subcore_barrier()

    @pl.when(c == 0)
    def _():
        pltpu.sync_copy(acc.at[pl.ds(base_row, ROWS_PER_SUB)],
                        out_i.at[pl.ds(base_row, ROWS_PER_SUB)])

        @pl.when(s == 0)
        def _():
            pltpu.sync_copy(acc.at[pl.ds(NS * ROWS_PER_SUB, ROWS_TAIL)],
                            out_i.at[pl.ds(NS * ROWS_PER_SUB, ROWS_TAIL)])

    @pl.when(c == 1)
    def _():
        pltpu.sync_copy(acc.at[pl.ds(base_row, ROWS_PER_SUB)],
                        out_u.at[pl.ds(base_row, ROWS_PER_SUB)])

        @pl.when(s == 0)
        def _():
            pltpu.sync_copy(acc.at[pl.ds(NS * ROWS_PER_SUB, ROWS_TAIL)],
                            out_u.at[pl.ds(NS * ROWS_PER_SUB, ROWS_TAIL)])


@jax.jit
def kernel(user_emb, item_emb, edge_index, edge_norm):
    u = edge_index[0].astype(jnp.int32)
    i = edge_index[1].astype(jnp.int32)
    tab = jnp.concatenate([user_emb, item_emb], axis=0)
    gidx = jnp.stack([u, i + N_USERS], axis=0)
    sidx = jnp.stack([i, u], axis=0)

    mesh = plsc.VectorSubcoreMesh(core_axis_name="c", subcore_axis_name="s",
                                  num_cores=NC, num_subcores=NS)
    run = pl.kernel(
        _body,
        out_type=(jax.ShapeDtypeStruct((N_USERS, D), jnp.float32),
                  jax.ShapeDtypeStruct((N_ITEMS, D), jnp.float32)),
        mesh=mesh,
        compiler_params=pltpu.CompilerParams(needs_layout_passes=False),
        scratch_types=[
            pltpu.VMEM_SHARED((N_USERS, D), jnp.float32),
            pltpu.VMEM((BLK,), jnp.int32),
            pltpu.VMEM((BLK,), jnp.int32),
            pltpu.VMEM((BLK,), jnp.int32),
            pltpu.VMEM((BLK,), jnp.int32),
            pltpu.VMEM((BLK,), jnp.float32),
            pltpu.VMEM((BLK,), jnp.float32),
            pltpu.VMEM((BLK, D), jnp.float32),
            pltpu.VMEM((BLK, D), jnp.float32),
            pltpu.SemaphoreType.DMA,
            pltpu.SemaphoreType.DMA,
        ],
    )
    agg_users, agg_items = run(tab, gidx, sidx, edge_norm)
    return (agg_users, agg_items)
